# Initial kernel scaffold; baseline (speedup 1.0000x reference)
import jax
import jax.numpy as jnp
from jax.experimental import pallas as pl


def kernel(x, Wg, Wr, W1, W2):
    raise NotImplementedError("write your pallas kernel here")


# speedup vs baseline: 3.3635x; 3.3635x over previous
import functools

import jax
import jax.numpy as jnp
from jax import lax
from jax.experimental import pallas as pl
from jax.experimental.pallas import tpu as pltpu
from jax.experimental.pallas import tpu_sc as plsc

T = 2048
D = 1024
DFF = 4096
G = 4
EG = 4
NE = G * EG
NA = 2 * T
BLK = 128
NBLK = 48
NPAD = NBLK * BLK
NW = 32


def _router_body(xf_ref, wcat_ref, gl_ref, ent_ref, gw_ref, pos_ref,
                 be_ref, nact_ref):
    xf = xf_ref[...]
    logits = lax.dot_general(
        xf.astype(jnp.bfloat16), wcat_ref[...].astype(jnp.bfloat16),
        (((1,), (1,)), ((), ())),
        preferred_element_type=jnp.float32)
    gl = logits[:, :G]
    gl_ref[...] = gl

    m = jnp.max(gl, axis=1, keepdims=True)
    ex = jnp.exp(gl - m)
    z = jnp.sum(ex, axis=1, keepdims=True)
    p = ex / z
    logp = (gl - m) - jnp.log(z)
    ent_ref[...] = jnp.reshape(-jnp.sum(p * logp) / T, (1, 1))

    ids4 = lax.broadcasted_iota(jnp.int32, (T, G), 1)
    a1 = jnp.min(jnp.where(gl == m, ids4, G), axis=1, keepdims=True)
    masked = jnp.where(ids4 == a1, -jnp.inf, gl)
    m2 = jnp.max(masked, axis=1, keepdims=True)
    a2 = jnp.min(jnp.where(masked == m2, ids4, G), axis=1, keepdims=True)
    e2v = jnp.exp(m2 - m)
    gw0 = 1.0 / (1.0 + e2v)
    gw1 = e2v / (1.0 + e2v)
    gw_ref[...] = jnp.concatenate([gw0, gw1], axis=0)

    eids = []
    for g in range(G):
        elg = logits[:, G + EG * g: G + EG * (g + 1)]
        mm = jnp.max(elg, axis=1, keepdims=True)
        ag = jnp.min(jnp.where(elg == mm, ids4, EG), axis=1, keepdims=True)
        eids.append(ag.astype(jnp.float32))
    eidm = jnp.concatenate(eids, axis=1)
    sel1 = jnp.sum(jnp.where(ids4 == a1, eidm, 0.0), axis=1, keepdims=True)
    sel2 = jnp.sum(jnp.where(ids4 == a2, eidm, 0.0), axis=1, keepdims=True)
    e1 = a1.astype(jnp.float32) * EG + sel1
    e2 = a2.astype(jnp.float32) * EG + sel2

    ids16 = lax.broadcasted_iota(jnp.int32, (NA, NE), 1).astype(jnp.float32)
    ea = jnp.concatenate([e1, e2], axis=0)
    oh = (ea == ids16).astype(jnp.float32)
    incl = oh
    k = 1
    while k < NA:
        shifted = jnp.concatenate(
            [jnp.zeros((k, NE), jnp.float32), incl[:-k, :]], axis=0)
        incl = incl + shifted
        k *= 2
    rex = incl - oh
    counts = incl[NA - 1:NA, :]
    pc = jnp.ceil(counts / BLK) * BLK
    iot16r = lax.broadcasted_iota(jnp.int32, (NE, NE), 0)
    iot16c = lax.broadcasted_iota(jnp.int32, (NE, NE), 1)
    mstrict = (iot16r < iot16c).astype(jnp.float32)
    po = lax.dot_general(pc, mstrict, (((1,), (0,)), ((), ())),
                         preferred_element_type=jnp.float32)
    pos = jnp.sum((po + rex) * oh, axis=1, keepdims=True)
    pos_ref[...] = pos.astype(jnp.int32)

    bstart = (lax.broadcasted_iota(jnp.int32, (NBLK, NE), 0)
              .astype(jnp.float32) * BLK)
    be = jnp.sum((po <= bstart).astype(jnp.float32), axis=1, keepdims=True) - 1.0
    be_ref[...] = be.astype(jnp.int32)
    nact_ref[...] = jnp.reshape((jnp.sum(pc) / BLK).astype(jnp.int32), (1, 1))


def _router_call(xf, wcat):
    return pl.pallas_call(
        _router_body,
        out_shape=[
            jax.ShapeDtypeStruct((T, G), jnp.float32),
            jax.ShapeDtypeStruct((1, 1), jnp.float32),
            jax.ShapeDtypeStruct((NA, 1), jnp.float32),
            jax.ShapeDtypeStruct((NA, 1), jnp.int32),
            jax.ShapeDtypeStruct((NBLK, 1), jnp.int32),
            jax.ShapeDtypeStruct((1, 1), jnp.int32),
        ],
    )(xf, wcat)


def _dispatch_body(xf_hbm, pos_hbm, xg_hbm, pos_v, rows_v, sem):
    wid = lax.axis_index("s") * 2 + lax.axis_index("c")
    base = wid * (NA // NW)
    for o in (0, 64):
        start = base + o
        pltpu.sync_copy(pos_hbm.at[pl.ds(start, 64)], pos_v)
        pltpu.sync_copy(xf_hbm.at[pl.ds(start % T, 64)], rows_v)
        pltpu.async_copy(rows_v, xg_hbm.at[pos_v], sem).wait()


def _dispatch_call(xf, posf):
    mesh = plsc.VectorSubcoreMesh(core_axis_name="c", subcore_axis_name="s")
    fn = functools.partial(
        pl.kernel, mesh=mesh,
        out_type=jax.ShapeDtypeStruct((NPAD, D), jnp.float32),
        scratch_types=[
            pltpu.VMEM((64,), jnp.int32),
            pltpu.VMEM((64, D), jnp.float32),
            pltpu.SemaphoreType.DMA,
        ],
    )(_dispatch_body)
    return fn(xf, posf)


def _ffn_body(nact_ref, be_ref, xg_ref, w1_ref, w2_ref, y_ref):
    i = pl.program_id(0)

    @pl.when(i < nact_ref[0])
    def _():
        xb = xg_ref[...].astype(jnp.bfloat16)
        h = lax.dot_general(xb, w1_ref[0], (((1,), (1,)), ((), ())),
                            preferred_element_type=jnp.float32)
        h = 0.5 * h * (1.0 + lax.erf(h * 0.7071067811865476))
        y = lax.dot_general(h.astype(jnp.bfloat16), w2_ref[0],
                            (((1,), (1,)), ((), ())),
                            preferred_element_type=jnp.float32)
        y_ref[...] = y


def _ffn_call(nact, be, xg, w1b, w2b):
    grid_spec = pltpu.PrefetchScalarGridSpec(
        num_scalar_prefetch=2,
        grid=(NBLK,),
        in_specs=[
            pl.BlockSpec((BLK, D), lambda i, nact, be: (i, 0)),
            pl.BlockSpec((1, DFF, D), lambda i, nact, be: (be[i], 0, 0)),
            pl.BlockSpec((1, D, DFF), lambda i, nact, be: (be[i], 0, 0)),
        ],
        out_specs=pl.BlockSpec((BLK, D), lambda i, nact, be: (i, 0)),
    )
    return pl.pallas_call(
        _ffn_body,
        grid_spec=grid_spec,
        out_shape=jax.ShapeDtypeStruct((NPAD, D), jnp.float32),
    )(nact, be, xg, w1b, w2b)


def _combine_body(y_hbm, pos0_hbm, pos1_hbm, gw0_hbm, gw1_hbm, out_hbm,
                  p0_v, p1_v, g0_v, g1_v, y0_v, y1_v, sem):
    wid = lax.axis_index("s") * 2 + lax.axis_index("c")
    base = wid * (T // NW)
    for o in (0, 32):
        start = base + o
        pltpu.sync_copy(pos0_hbm.at[pl.ds(start, 32)], p0_v)
        pltpu.sync_copy(pos1_hbm.at[pl.ds(start, 32)], p1_v)
        pltpu.sync_copy(gw0_hbm.at[pl.ds(start, 32)], g0_v)
        pltpu.sync_copy(gw1_hbm.at[pl.ds(start, 32)], g1_v)
        pltpu.async_copy(y_hbm.at[p0_v], y0_v, sem).wait()
        pltpu.async_copy(y_hbm.at[p1_v], y1_v, sem).wait()

        for half in range(2):
            wv0 = g0_v[pl.ds(16 * half, 16)]
            wv1 = g1_v[pl.ds(16 * half, 16)]
            for l in range(16):
                t = 16 * half + l
                a = wv0[l]
                b = wv1[l]

                def lane_body(j, c, t=t, a=a, b=b):
                    sl = pl.ds(j * 16, 16)
                    y0_v[t, sl] = a * y0_v[t, sl] + b * y1_v[t, sl]
                    return c

                lax.fori_loop(0, D // 16, lane_body, 0)
        pltpu.sync_copy(y0_v, out_hbm.at[pl.ds(start, 32)])


def _combine_call(y, pos0, pos1, gw0, gw1):
    mesh = plsc.VectorSubcoreMesh(core_axis_name="c", subcore_axis_name="s")
    fn = functools.partial(
        pl.kernel, mesh=mesh,
        out_type=jax.ShapeDtypeStruct((T, D), jnp.float32),
        scratch_types=[
            pltpu.VMEM((32,), jnp.int32),
            pltpu.VMEM((32,), jnp.int32),
            pltpu.VMEM((32,), jnp.float32),
            pltpu.VMEM((32,), jnp.float32),
            pltpu.VMEM((32, D), jnp.float32),
            pltpu.VMEM((32, D), jnp.float32),
            pltpu.SemaphoreType.DMA,
        ],
    )(_combine_body)
    return fn(y, pos0, pos1, gw0, gw1)


def kernel(x, Wg, Wr, W1, W2):
    Bc, Tc, Dc = x.shape
    xf = x.reshape(Tc, Dc)
    wcat = jnp.concatenate([Wg, Wr.reshape(NE, D)], axis=0)

    gl, ent, gw, pos, be, nact = _router_call(xf, wcat)

    posf = pos.reshape(NA)
    gwf = gw.reshape(NA)
    bef = be.reshape(NBLK)
    nactf = nact.reshape(1)

    xg = _dispatch_call(xf, posf)
    w1b = W1.reshape(NE, DFF, D).astype(jnp.bfloat16)
    w2b = W2.reshape(NE, D, DFF).astype(jnp.bfloat16)
    y = _ffn_call(nactf, bef, xg, w1b, w2b)
    out = _combine_call(y, posf[:T], posf[T:], gwf[:T], gwf[T:])

    return (out.reshape(Bc, Tc, Dc), gl.reshape(Bc, Tc, G), ent[0, 0])


# speedup vs baseline: 3.4222x; 1.0174x over previous
import functools

import jax
import jax.numpy as jnp
from jax import lax
from jax.experimental import pallas as pl
from jax.experimental.pallas import tpu as pltpu
from jax.experimental.pallas import tpu_sc as plsc

T = 2048
D = 1024
DFF = 4096
G = 4
EG = 4
NE = G * EG
NA = 2 * T
BLK = 128
NBLK = 48
NPAD = NBLK * BLK
NW = 32


def _router_body(xf_ref, wcat_ref, gl_ref, ent_ref, gw_ref, pos_ref,
                 be_ref, nact_ref):
    xf = xf_ref[...]
    logits = lax.dot_general(
        xf.astype(jnp.bfloat16), wcat_ref[...].astype(jnp.bfloat16),
        (((1,), (1,)), ((), ())),
        preferred_element_type=jnp.float32)
    gl = logits[:, :G]
    gl_ref[...] = gl

    m = jnp.max(gl, axis=1, keepdims=True)
    ex = jnp.exp(gl - m)
    z = jnp.sum(ex, axis=1, keepdims=True)
    p = ex / z
    logp = (gl - m) - jnp.log(z)
    ent_ref[...] = jnp.reshape(-jnp.sum(p * logp) / T, (1, 1))

    ids4 = lax.broadcasted_iota(jnp.int32, (T, G), 1)
    a1 = jnp.min(jnp.where(gl == m, ids4, G), axis=1, keepdims=True)
    masked = jnp.where(ids4 == a1, -jnp.inf, gl)
    m2 = jnp.max(masked, axis=1, keepdims=True)
    a2 = jnp.min(jnp.where(masked == m2, ids4, G), axis=1, keepdims=True)
    e2v = jnp.exp(m2 - m)
    gw0 = 1.0 / (1.0 + e2v)
    gw1 = e2v / (1.0 + e2v)
    gw_ref[...] = jnp.concatenate([gw0, gw1], axis=0)

    eids = []
    for g in range(G):
        elg = logits[:, G + EG * g: G + EG * (g + 1)]
        mm = jnp.max(elg, axis=1, keepdims=True)
        ag = jnp.min(jnp.where(elg == mm, ids4, EG), axis=1, keepdims=True)
        eids.append(ag.astype(jnp.float32))
    eidm = jnp.concatenate(eids, axis=1)
    sel1 = jnp.sum(jnp.where(ids4 == a1, eidm, 0.0), axis=1, keepdims=True)
    sel2 = jnp.sum(jnp.where(ids4 == a2, eidm, 0.0), axis=1, keepdims=True)
    e1 = a1.astype(jnp.float32) * EG + sel1
    e2 = a2.astype(jnp.float32) * EG + sel2

    ids16 = lax.broadcasted_iota(jnp.int32, (NA, NE), 1).astype(jnp.float32)
    ea = jnp.concatenate([e1, e2], axis=0)
    oh = (ea == ids16).astype(jnp.float32)
    incl = oh
    k = 1
    while k < NA:
        shifted = jnp.concatenate(
            [jnp.zeros((k, NE), jnp.float32), incl[:-k, :]], axis=0)
        incl = incl + shifted
        k *= 2
    rex = incl - oh
    counts = incl[NA - 1:NA, :]
    pc = jnp.ceil(counts / BLK) * BLK
    iot16r = lax.broadcasted_iota(jnp.int32, (NE, NE), 0)
    iot16c = lax.broadcasted_iota(jnp.int32, (NE, NE), 1)
    mstrict = (iot16r < iot16c).astype(jnp.float32)
    po = lax.dot_general(pc, mstrict, (((1,), (0,)), ((), ())),
                         preferred_element_type=jnp.float32)
    pos = jnp.sum((po + rex) * oh, axis=1, keepdims=True)
    pos_ref[...] = pos.astype(jnp.int32)

    bstart = (lax.broadcasted_iota(jnp.int32, (NBLK, NE), 0)
              .astype(jnp.float32) * BLK)
    be = jnp.sum((po <= bstart).astype(jnp.float32), axis=1, keepdims=True) - 1.0
    be_ref[...] = be.astype(jnp.int32)
    nact_ref[...] = jnp.reshape((jnp.sum(pc) / BLK).astype(jnp.int32), (1, 1))


def _router_call(xf, wcat):
    return pl.pallas_call(
        _router_body,
        out_shape=[
            jax.ShapeDtypeStruct((T, G), jnp.float32),
            jax.ShapeDtypeStruct((1, 1), jnp.float32),
            jax.ShapeDtypeStruct((NA, 1), jnp.float32),
            jax.ShapeDtypeStruct((NA, 1), jnp.int32),
            jax.ShapeDtypeStruct((NBLK, 1), jnp.int32),
            jax.ShapeDtypeStruct((1, 1), jnp.int32),
        ],
    )(xf, wcat)


def _dispatch_body(xf_hbm, pos_hbm, xg_hbm, pos_v, rows_v, sem):
    wid = lax.axis_index("s") * 2 + lax.axis_index("c")
    base = wid * (NA // NW)
    for o in (0, 64):
        start = base + o
        pltpu.sync_copy(pos_hbm.at[pl.ds(start, 64)], pos_v)
        pltpu.sync_copy(xf_hbm.at[pl.ds(start % T, 64)], rows_v)
        pltpu.async_copy(rows_v, xg_hbm.at[pos_v], sem).wait()


def _dispatch_call(xf, posf):
    mesh = plsc.VectorSubcoreMesh(core_axis_name="c", subcore_axis_name="s")
    fn = functools.partial(
        pl.kernel, mesh=mesh,
        out_type=jax.ShapeDtypeStruct((NPAD, D), jnp.float32),
        scratch_types=[
            pltpu.VMEM((64,), jnp.int32),
            pltpu.VMEM((64, D), jnp.float32),
            pltpu.SemaphoreType.DMA,
        ],
    )(_dispatch_body)
    return fn(xf, posf)


DFFH = DFF // 2


def _ffn_body(nact_ref, be_ref, xg_ref, w1_ref, w2_ref, y_ref):
    i = pl.program_id(0)
    k = pl.program_id(1)

    @pl.when(i < nact_ref[0])
    def _():
        xb = xg_ref[...].astype(jnp.bfloat16)
        h = lax.dot_general(xb, w1_ref[0].astype(jnp.bfloat16),
                            (((1,), (1,)), ((), ())),
                            preferred_element_type=jnp.float32)
        h = 0.5 * h * (1.0 + lax.erf(h * 0.7071067811865476))
        y = lax.dot_general(h.astype(jnp.bfloat16),
                            w2_ref[0].astype(jnp.bfloat16),
                            (((1,), (1,)), ((), ())),
                            preferred_element_type=jnp.float32)

        @pl.when(k == 0)
        def _():
            y_ref[...] = y

        @pl.when(k != 0)
        def _():
            y_ref[...] += y


def _ffn_call(nact, be, xg, w1f, w2f):
    grid_spec = pltpu.PrefetchScalarGridSpec(
        num_scalar_prefetch=2,
        grid=(NBLK, 2),
        in_specs=[
            pl.BlockSpec((BLK, D), lambda i, k, nact, be: (i, 0)),
            pl.BlockSpec((1, DFFH, D), lambda i, k, nact, be: (be[i], k, 0)),
            pl.BlockSpec((1, D, DFFH), lambda i, k, nact, be: (be[i], 0, k)),
        ],
        out_specs=pl.BlockSpec((BLK, D), lambda i, k, nact, be: (i, 0)),
    )
    return pl.pallas_call(
        _ffn_body,
        grid_spec=grid_spec,
        out_shape=jax.ShapeDtypeStruct((NPAD, D), jnp.float32),
    )(nact, be, xg, w1f, w2f)


def _combine_body(y_hbm, pos0_hbm, pos1_hbm, gw0_hbm, gw1_hbm, out_hbm,
                  p0_v, p1_v, g0_v, g1_v, y0_v, y1_v, sem):
    wid = lax.axis_index("s") * 2 + lax.axis_index("c")
    base = wid * (T // NW)
    for o in (0, 32):
        start = base + o
        pltpu.sync_copy(pos0_hbm.at[pl.ds(start, 32)], p0_v)
        pltpu.sync_copy(pos1_hbm.at[pl.ds(start, 32)], p1_v)
        pltpu.sync_copy(gw0_hbm.at[pl.ds(start, 32)], g0_v)
        pltpu.sync_copy(gw1_hbm.at[pl.ds(start, 32)], g1_v)
        pltpu.async_copy(y_hbm.at[p0_v], y0_v, sem).wait()
        pltpu.async_copy(y_hbm.at[p1_v], y1_v, sem).wait()

        for half in range(2):
            wv0 = g0_v[pl.ds(16 * half, 16)]
            wv1 = g1_v[pl.ds(16 * half, 16)]
            for l in range(16):
                t = 16 * half + l
                a = wv0[l]
                b = wv1[l]

                def lane_body(j, c, t=t, a=a, b=b):
                    sl = pl.ds(j * 16, 16)
                    y0_v[t, sl] = a * y0_v[t, sl] + b * y1_v[t, sl]
                    return c

                lax.fori_loop(0, D // 16, lane_body, 0)
        pltpu.sync_copy(y0_v, out_hbm.at[pl.ds(start, 32)])


def _combine_call(y, pos0, pos1, gw0, gw1):
    mesh = plsc.VectorSubcoreMesh(core_axis_name="c", subcore_axis_name="s")
    fn = functools.partial(
        pl.kernel, mesh=mesh,
        out_type=jax.ShapeDtypeStruct((T, D), jnp.float32),
        scratch_types=[
            pltpu.VMEM((32,), jnp.int32),
            pltpu.VMEM((32,), jnp.int32),
            pltpu.VMEM((32,), jnp.float32),
            pltpu.VMEM((32,), jnp.float32),
            pltpu.VMEM((32, D), jnp.float32),
            pltpu.VMEM((32, D), jnp.float32),
            pltpu.SemaphoreType.DMA,
        ],
    )(_combine_body)
    return fn(y, pos0, pos1, gw0, gw1)


def kernel(x, Wg, Wr, W1, W2):
    Bc, Tc, Dc = x.shape
    xf = x.reshape(Tc, Dc)
    wcat = jnp.concatenate([Wg, Wr.reshape(NE, D)], axis=0)

    gl, ent, gw, pos, be, nact = _router_call(xf, wcat)

    posf = pos.reshape(NA)
    gwf = gw.reshape(NA)
    bef = be.reshape(NBLK)
    nactf = nact.reshape(1)

    xg = _dispatch_call(xf, posf)
    y = _ffn_call(nactf, bef, xg,
                  W1.reshape(NE, DFF, D), W2.reshape(NE, D, DFF))
    out = _combine_call(y, posf[:T], posf[T:], gwf[:T], gwf[T:])

    return (out.reshape(Bc, Tc, Dc), gl.reshape(Bc, Tc, G), ent[0, 0])


# speedup vs baseline: 4.3316x; 1.2658x over previous
import functools

import jax
import jax.numpy as jnp
from jax import lax
from jax.experimental import pallas as pl
from jax.experimental.pallas import tpu as pltpu
from jax.experimental.pallas import tpu_sc as plsc

T = 2048
D = 1024
DFF = 4096
G = 4
EG = 4
NE = G * EG
NA = 2 * T
BLK = 128
NBLK = 48
NPAD = NBLK * BLK
NW = 32


def _router_body(xf_ref, wcat_ref, gl_ref, ent_ref, gw_ref, pos_ref,
                 be_ref, nact_ref):
    xf = xf_ref[...]
    logits = lax.dot_general(
        xf.astype(jnp.bfloat16), wcat_ref[...].astype(jnp.bfloat16),
        (((1,), (1,)), ((), ())),
        preferred_element_type=jnp.float32)
    gl = logits[:, :G]
    gl_ref[...] = gl

    m = jnp.max(gl, axis=1, keepdims=True)
    ex = jnp.exp(gl - m)
    z = jnp.sum(ex, axis=1, keepdims=True)
    p = ex / z
    logp = (gl - m) - jnp.log(z)
    ent_ref[...] = jnp.reshape(-jnp.sum(p * logp) / T, (1, 1))

    ids4 = lax.broadcasted_iota(jnp.int32, (T, G), 1)
    a1 = jnp.min(jnp.where(gl == m, ids4, G), axis=1, keepdims=True)
    masked = jnp.where(ids4 == a1, -jnp.inf, gl)
    m2 = jnp.max(masked, axis=1, keepdims=True)
    a2 = jnp.min(jnp.where(masked == m2, ids4, G), axis=1, keepdims=True)
    e2v = jnp.exp(m2 - m)
    gw0 = 1.0 / (1.0 + e2v)
    gw1 = e2v / (1.0 + e2v)
    gw_ref[...] = jnp.concatenate([gw0, gw1], axis=0)

    eids = []
    for g in range(G):
        elg = logits[:, G + EG * g: G + EG * (g + 1)]
        mm = jnp.max(elg, axis=1, keepdims=True)
        ag = jnp.min(jnp.where(elg == mm, ids4, EG), axis=1, keepdims=True)
        eids.append(ag.astype(jnp.float32))
    eidm = jnp.concatenate(eids, axis=1)
    sel1 = jnp.sum(jnp.where(ids4 == a1, eidm, 0.0), axis=1, keepdims=True)
    sel2 = jnp.sum(jnp.where(ids4 == a2, eidm, 0.0), axis=1, keepdims=True)
    e1 = a1.astype(jnp.float32) * EG + sel1
    e2 = a2.astype(jnp.float32) * EG + sel2

    ids16 = lax.broadcasted_iota(jnp.int32, (NA, NE), 1).astype(jnp.float32)
    ea = jnp.concatenate([e1, e2], axis=0)
    oh = (ea == ids16).astype(jnp.float32)
    incl = oh
    k = 1
    while k < NA:
        shifted = jnp.concatenate(
            [jnp.zeros((k, NE), jnp.float32), incl[:-k, :]], axis=0)
        incl = incl + shifted
        k *= 2
    rex = incl - oh
    counts = incl[NA - 1:NA, :]
    pc = jnp.ceil(counts / BLK) * BLK
    iot16r = lax.broadcasted_iota(jnp.int32, (NE, NE), 0)
    iot16c = lax.broadcasted_iota(jnp.int32, (NE, NE), 1)
    mstrict = (iot16r < iot16c).astype(jnp.float32)
    po = lax.dot_general(pc, mstrict, (((1,), (0,)), ((), ())),
                         preferred_element_type=jnp.float32)
    pos = jnp.sum((po + rex) * oh, axis=1, keepdims=True)
    pos_ref[...] = pos.astype(jnp.int32)

    bstart = (lax.broadcasted_iota(jnp.int32, (NBLK, NE), 0)
              .astype(jnp.float32) * BLK)
    be = jnp.sum((po <= bstart).astype(jnp.float32), axis=1, keepdims=True) - 1.0
    be_ref[...] = be.astype(jnp.int32)
    nact_ref[...] = jnp.reshape((jnp.sum(pc) / BLK).astype(jnp.int32), (1, 1))


def _router_call(xf, wcat):
    return pl.pallas_call(
        _router_body,
        out_shape=[
            jax.ShapeDtypeStruct((T, G), jnp.float32),
            jax.ShapeDtypeStruct((1, 1), jnp.float32),
            jax.ShapeDtypeStruct((NA, 1), jnp.float32),
            jax.ShapeDtypeStruct((NA, 1), jnp.int32),
            jax.ShapeDtypeStruct((NBLK, 1), jnp.int32),
            jax.ShapeDtypeStruct((1, 1), jnp.int32),
        ],
    )(xf, wcat)


def _dispatch_body(xf_hbm, pos_hbm, xg_hbm, pos_v, rows_v, sem):
    wid = lax.axis_index("s") * 2 + lax.axis_index("c")
    base = wid * (NA // NW)
    for o in (0, 64):
        start = base + o
        pltpu.sync_copy(pos_hbm.at[pl.ds(start, 64)], pos_v)
        pltpu.sync_copy(xf_hbm.at[pl.ds(start % T, 64)], rows_v)
        pltpu.async_copy(rows_v, xg_hbm.at[pos_v], sem).wait()


def _dispatch_call(xf, posf):
    mesh = plsc.VectorSubcoreMesh(core_axis_name="c", subcore_axis_name="s")
    fn = functools.partial(
        pl.kernel, mesh=mesh,
        out_type=jax.ShapeDtypeStruct((NPAD, D), jnp.float32),
        scratch_types=[
            pltpu.VMEM((64,), jnp.int32),
            pltpu.VMEM((64, D), jnp.float32),
            pltpu.SemaphoreType.DMA,
        ],
    )(_dispatch_body)
    return fn(xf, posf)


DFFH = DFF // 2


def _ffn_body(nact_ref, be_ref, xg_ref, w1_ref, w2_ref, y_ref, acc_ref):
    k = pl.program_id(0)
    i = pl.program_id(1)

    @pl.when(i < nact_ref[0])
    def _():
        xb = xg_ref[...].astype(jnp.bfloat16)
        h = lax.dot_general(xb, w1_ref[0].astype(jnp.bfloat16),
                            (((1,), (1,)), ((), ())),
                            preferred_element_type=jnp.float32)
        h = 0.5 * h * (1.0 + lax.erf(h * 0.7071067811865476))
        y = lax.dot_general(h.astype(jnp.bfloat16),
                            w2_ref[0].astype(jnp.bfloat16),
                            (((1,), (1,)), ((), ())),
                            preferred_element_type=jnp.float32)

        @pl.when(k == 0)
        def _():
            acc_ref[pl.ds(i * BLK, BLK), :] = y.astype(jnp.bfloat16)

        @pl.when(k == 1)
        def _():
            y_ref[...] = acc_ref[pl.ds(i * BLK, BLK), :].astype(jnp.float32) + y


def _ffn_call(nact, be, xg, w1f, w2f):
    grid_spec = pltpu.PrefetchScalarGridSpec(
        num_scalar_prefetch=2,
        grid=(2, NBLK),
        in_specs=[
            pl.BlockSpec((BLK, D), lambda k, i, nact, be: (i, 0)),
            pl.BlockSpec((1, DFFH, D), lambda k, i, nact, be: (be[i], k, 0)),
            pl.BlockSpec((1, D, DFFH), lambda k, i, nact, be: (be[i], 0, k)),
        ],
        out_specs=pl.BlockSpec(
            (BLK, D), lambda k, i, nact, be: (jnp.where(k == 1, i, 0), 0)),
        scratch_shapes=[pltpu.VMEM((NPAD, D), jnp.bfloat16)],
    )
    return pl.pallas_call(
        _ffn_body,
        grid_spec=grid_spec,
        out_shape=jax.ShapeDtypeStruct((NPAD, D), jnp.float32),
    )(nact, be, xg, w1f, w2f)


def _combine_body(y_hbm, pos0_hbm, pos1_hbm, gw0_hbm, gw1_hbm, out_hbm,
                  p0_v, p1_v, g0_v, g1_v, y0_v, y1_v, sem):
    wid = lax.axis_index("s") * 2 + lax.axis_index("c")
    base = wid * (T // NW)
    for o in (0, 32):
        start = base + o
        pltpu.sync_copy(pos0_hbm.at[pl.ds(start, 32)], p0_v)
        pltpu.sync_copy(pos1_hbm.at[pl.ds(start, 32)], p1_v)
        pltpu.sync_copy(gw0_hbm.at[pl.ds(start, 32)], g0_v)
        pltpu.sync_copy(gw1_hbm.at[pl.ds(start, 32)], g1_v)
        pltpu.async_copy(y_hbm.at[p0_v], y0_v, sem).wait()
        pltpu.async_copy(y_hbm.at[p1_v], y1_v, sem).wait()

        for half in range(2):
            wv0 = g0_v[pl.ds(16 * half, 16)]
            wv1 = g1_v[pl.ds(16 * half, 16)]
            for l in range(16):
                t = 16 * half + l
                a = wv0[l]
                b = wv1[l]

                def lane_body(j, c, t=t, a=a, b=b):
                    sl = pl.ds(j * 16, 16)
                    y0_v[t, sl] = a * y0_v[t, sl] + b * y1_v[t, sl]
                    return c

                lax.fori_loop(0, D // 16, lane_body, 0)
        pltpu.sync_copy(y0_v, out_hbm.at[pl.ds(start, 32)])


def _combine_call(y, pos0, pos1, gw0, gw1):
    mesh = plsc.VectorSubcoreMesh(core_axis_name="c", subcore_axis_name="s")
    fn = functools.partial(
        pl.kernel, mesh=mesh,
        out_type=jax.ShapeDtypeStruct((T, D), jnp.float32),
        scratch_types=[
            pltpu.VMEM((32,), jnp.int32),
            pltpu.VMEM((32,), jnp.int32),
            pltpu.VMEM((32,), jnp.float32),
            pltpu.VMEM((32,), jnp.float32),
            pltpu.VMEM((32, D), jnp.float32),
            pltpu.VMEM((32, D), jnp.float32),
            pltpu.SemaphoreType.DMA,
        ],
    )(_combine_body)
    return fn(y, pos0, pos1, gw0, gw1)


def kernel(x, Wg, Wr, W1, W2):
    Bc, Tc, Dc = x.shape
    xf = x.reshape(Tc, Dc)
    wcat = jnp.concatenate([Wg, Wr.reshape(NE, D)], axis=0)

    gl, ent, gw, pos, be, nact = _router_call(xf, wcat)

    posf = pos.reshape(NA)
    gwf = gw.reshape(NA)
    bef = be.reshape(NBLK)
    nactf = nact.reshape(1)

    xg = _dispatch_call(xf, posf)
    y = _ffn_call(nactf, bef, xg,
                  W1.reshape(NE, DFF, D), W2.reshape(NE, D, DFF))
    out = _combine_call(y, posf[:T], posf[T:], gwf[:T], gwf[T:])

    return (out.reshape(Bc, Tc, Dc), gl.reshape(Bc, Tc, G), ent[0, 0])
